# baseline (speedup 1.0000x reference)
import jax
import jax.numpy as jnp
from jax import lax
from jax.experimental import pallas as pl
from jax.experimental.pallas import tpu as pltpu
from jax.experimental.pallas import tpu_sc as plsc

_N = 10000
_E = 160000
_D = 256
_H = 256
_HALF = 128
_EPS = 1e-5
_NC = 2
_NS = 16
_CH = 128
_EPAD = 163840
_PT_PROP = _EPAD // _NS
_NP = 10240
_RPT = _NP // _NS
_NACC = _NP
_TRASH = _N


_SC_CACHE = {}


def _sc_mesh():
    return plsc.VectorSubcoreMesh(
        core_axis_name="c", subcore_axis_name="s",
        num_cores=_NC, num_subcores=_NS)


_DEPTH = 2
_NCHUNK = _PT_PROP // _CH


def _prop_body(u_hbm, src_hbm, dst_hbm, g_hbm, idxs_v, idxd_v, rows_v, acc_sh,
               *sems):
    c = lax.axis_index("c")
    s = lax.axis_index("s")
    roff = c * _NP
    base = s * _PT_PROP

    def load_and_gather(j, b):
        off = base + j * _CH
        pltpu.sync_copy(src_hbm.at[pl.ds(off, _CH)], idxs_v.at[b])
        pltpu.sync_copy(dst_hbm.at[pl.ds(off, _CH)], idxd_v.at[b])
        for t in range(_CH // 16):
            idxs_v[b, pl.ds(t * 16, 16)] = idxs_v[b, pl.ds(t * 16, 16)] + roff
        pltpu.async_copy(u_hbm.at[idxs_v.at[b]], rows_v.at[b], sems[b])

    def scatter_issue(b):
        pltpu.make_async_copy(u_hbm.at[idxs_v.at[b]], rows_v.at[b],
                              sems[b]).wait()
        pltpu.async_copy(rows_v.at[b], acc_sh.at[idxd_v.at[b]],
                         sems[_DEPTH + b], add=True)

    def scatter_wait(b):
        pltpu.make_async_copy(rows_v.at[b], acc_sh.at[idxd_v.at[b]],
                              sems[_DEPTH + b]).wait()

    for b in range(_DEPTH):
        load_and_gather(b, b)
    pltpu.sync_copy(u_hbm.at[pl.ds(roff + s * _RPT, _RPT)],
                    acc_sh.at[pl.ds(s * _RPT, _RPT)])
    plsc.subcore_barrier()

    def pair(jp, _):
        for b in range(_DEPTH):
            scatter_issue(b)
        for b in range(_DEPTH):
            scatter_wait(b)
            load_and_gather(_DEPTH * jp + b + _DEPTH, b)
        return 0

    lax.fori_loop(0, _NCHUNK // _DEPTH - 1, pair, 0)
    for b in range(_DEPTH):
        scatter_issue(b)
    for b in range(_DEPTH):
        scatter_wait(b)

    plsc.subcore_barrier()
    pltpu.sync_copy(acc_sh.at[pl.ds(s * _RPT, _RPT)],
                    g_hbm.at[c, pl.ds(s * _RPT, _RPT)])


def _prop(u, src_p, dst_p):
    if "prop" not in _SC_CACHE:
        _SC_CACHE["prop"] = pl.kernel(
            _prop_body,
            out_type=jax.ShapeDtypeStruct((_NC, _NP, _HALF), jnp.float32),
            mesh=_sc_mesh(),
            scratch_types=[
                pltpu.VMEM((_DEPTH, _CH), jnp.int32),
                pltpu.VMEM((_DEPTH, _CH), jnp.int32),
                pltpu.VMEM((_DEPTH, _CH, _HALF), jnp.float32),
                pltpu.VMEM_SHARED((_NACC, _HALF), jnp.float32),
            ] + [pltpu.SemaphoreType.DMA] * (2 * _DEPTH),
        )
    return _SC_CACHE["prop"](u, src_p, dst_p)


_R = 1000
_GRID = _N // _R


def _init_body(x_ref, w_ref, b_ref, gdeg_ref, u_ref, dinv_ref, dinv2_ref):
    h = jnp.dot(x_ref[...], w_ref[...],
                preferred_element_type=jnp.float32) + b_ref[...]
    degsl = gdeg_ref[0]
    dinv128 = lax.rsqrt(degsl)
    dinv_ref[...] = dinv128[:, :16]
    dinv2_ref[...] = (1.0 / degsl)[:, :16]
    u = h * dinv128[:, 0:1]
    u_ref[0] = u[:, :_HALF]
    u_ref[1] = u[:, _HALF:]


_init = pl.pallas_call(
    _init_body,
    grid=(_GRID,),
    in_specs=[
        pl.BlockSpec((_R, _D), lambda i: (i, 0)),
        pl.BlockSpec((_D, _H), lambda i: (0, 0)),
        pl.BlockSpec((1, _H), lambda i: (0, 0)),
        pl.BlockSpec((1, _R, _HALF), lambda i: (0, i, 0)),
    ],
    out_specs=[
        pl.BlockSpec((_NC, _R, _HALF), lambda i: (0, i, 0)),
        pl.BlockSpec((_R, 16), lambda i: (i, 0)),
        pl.BlockSpec((_R, 16), lambda i: (i, 0)),
    ],
    out_shape=[
        jax.ShapeDtypeStruct((_NC, _NP, _HALF), jnp.float32),
        jax.ShapeDtypeStruct((_N, 16), jnp.float32),
        jax.ShapeDtypeStruct((_N, 16), jnp.float32),
    ],
    compiler_params=pltpu.CompilerParams(dimension_semantics=("arbitrary",)),
)


def _hop_common(glo_ref, ghi_ref, dinv_ref, w_ref, out_ref, cs_ref, cq_ref):
    i = pl.program_id(0)
    g = jnp.concatenate([glo_ref[0], ghi_ref[0]], axis=1)
    y = g * dinv_ref[...][:, 0:1]
    o = jnp.dot(y, w_ref[...], preferred_element_type=jnp.float32)
    out_ref[...] = o

    @pl.when(i == 0)
    def _():
        cs_ref[...] = jnp.zeros_like(cs_ref)
        cq_ref[...] = jnp.zeros_like(cq_ref)

    cs_ref[...] += jnp.sum(o, axis=0, keepdims=True)
    cq_ref[...] += jnp.sum(o * o, axis=0, keepdims=True)
    return g


def _hop_body(glo_ref, ghi_ref, dinv_ref, dinv2_ref, w_ref, out_ref, un_ref,
              cs_ref, cq_ref):
    g = _hop_common(glo_ref, ghi_ref, dinv_ref, w_ref, out_ref, cs_ref, cq_ref)
    un = g * dinv2_ref[...][:, 0:1]
    un_ref[0] = un[:, :_HALF]
    un_ref[1] = un[:, _HALF:]


def _hop_body_nou(glo_ref, ghi_ref, dinv_ref, dinv2_ref, w_ref, out_ref,
                  cs_ref, cq_ref):
    _hop_common(glo_ref, ghi_ref, dinv_ref, w_ref, out_ref, cs_ref, cq_ref)


_HOP_IN_SPECS = [
    pl.BlockSpec((1, _R, _HALF), lambda i: (0, i, 0)),
    pl.BlockSpec((1, _R, _HALF), lambda i: (1, i, 0)),
    pl.BlockSpec((_R, 16), lambda i: (i, 0)),
    pl.BlockSpec((_R, 16), lambda i: (i, 0)),
    pl.BlockSpec((_H, _H), lambda i: (0, 0)),
]
_STAT_SPEC = pl.BlockSpec((1, _H), lambda i: (0, 0))
_STAT_SHAPE = jax.ShapeDtypeStruct((1, _H), jnp.float32)

_hop = pl.pallas_call(
    _hop_body,
    grid=(_GRID,),
    in_specs=_HOP_IN_SPECS,
    out_specs=[
        pl.BlockSpec((_R, _H), lambda i: (i, 0)),
        pl.BlockSpec((_NC, _R, _HALF), lambda i: (0, i, 0)),
        _STAT_SPEC,
        _STAT_SPEC,
    ],
    out_shape=[
        jax.ShapeDtypeStruct((_N, _H), jnp.float32),
        jax.ShapeDtypeStruct((_NC, _NP, _HALF), jnp.float32),
        _STAT_SHAPE,
        _STAT_SHAPE,
    ],
    compiler_params=pltpu.CompilerParams(dimension_semantics=("arbitrary",)),
)

_hop_nou = pl.pallas_call(
    _hop_body_nou,
    grid=(_GRID,),
    in_specs=_HOP_IN_SPECS,
    out_specs=[
        pl.BlockSpec((_R, _H), lambda i: (i, 0)),
        _STAT_SPEC,
        _STAT_SPEC,
    ],
    out_shape=[
        jax.ShapeDtypeStruct((_N, _H), jnp.float32),
        _STAT_SHAPE,
        _STAT_SHAPE,
    ],
    compiler_params=pltpu.CompilerParams(dimension_semantics=("arbitrary",)),
)


def _bn_common(o_refs, cs_refs, cq_refs, gam_ref, bet_ref, w3_ref, wb_ref):
    gam = gam_ref[...]
    bet = bet_ref[...]
    acc = jnp.zeros((_R, _H), jnp.float32) + wb_ref[...]
    for k in range(3):
        mu = cs_refs[k][...] / _N
        var = cq_refs[k][...] / _N - mu * mu
        sc = gam[k:k + 1] * lax.rsqrt(var + _EPS)
        shift = bet[k:k + 1] - mu * sc
        hn = o_refs[k][...] * sc + shift
        acc += jnp.dot(hn, w3_ref[k], preferred_element_type=jnp.float32)
    return jnp.maximum(acc, 0.0)


def _bn_body(o1, o2, o3, cs1, cs2, cs3, cq1, cq2, cq3, gam, bet, w3, wb,
             dinv_ref, h_ref, un_ref):
    h = _bn_common((o1, o2, o3), (cs1, cs2, cs3), (cq1, cq2, cq3), gam, bet,
                   w3, wb)
    h_ref[...] = h
    un = h * dinv_ref[...][:, 0:1]
    un_ref[0] = un[:, :_HALF]
    un_ref[1] = un[:, _HALF:]


def _bn_body_nou(o1, o2, o3, cs1, cs2, cs3, cq1, cq2, cq3, gam, bet, w3, wb,
                 h_ref):
    h_ref[...] = _bn_common((o1, o2, o3), (cs1, cs2, cs3), (cq1, cq2, cq3),
                            gam, bet, w3, wb)


_BN_IN_SPECS = (
    [pl.BlockSpec((_R, _H), lambda i: (i, 0))] * 3
    + [_STAT_SPEC] * 6
    + [
        pl.BlockSpec((3, _H), lambda i: (0, 0)),
        pl.BlockSpec((3, _H), lambda i: (0, 0)),
        pl.BlockSpec((3, _H, _H), lambda i: (0, 0, 0)),
        pl.BlockSpec((1, _H), lambda i: (0, 0)),
    ]
)

_bn = pl.pallas_call(
    _bn_body,
    grid=(_GRID,),
    in_specs=_BN_IN_SPECS + [pl.BlockSpec((_R, 16), lambda i: (i, 0))],
    out_specs=[
        pl.BlockSpec((_R, _H), lambda i: (i, 0)),
        pl.BlockSpec((_NC, _R, _HALF), lambda i: (0, i, 0)),
    ],
    out_shape=[
        jax.ShapeDtypeStruct((_N, _H), jnp.float32),
        jax.ShapeDtypeStruct((_NC, _NP, _HALF), jnp.float32),
    ],
    compiler_params=pltpu.CompilerParams(dimension_semantics=("arbitrary",)),
)

_bn_nou = pl.pallas_call(
    _bn_body_nou,
    grid=(_GRID,),
    in_specs=_BN_IN_SPECS,
    out_specs=[pl.BlockSpec((_R, _H), lambda i: (i, 0))],
    out_shape=[jax.ShapeDtypeStruct((_N, _H), jnp.float32)],
    compiler_params=pltpu.CompilerParams(dimension_semantics=("arbitrary",)),
)


def kernel(x, edge_index, fc_w, fc_b, conv0_w1, conv0_w2, conv0_w3, bn0_gamma,
           bn0_beta, conv1_w1, conv1_w2, conv1_w3, bn1_gamma, bn1_beta, W_w,
           W_b):
    pad = _EPAD - _E
    src_p = jnp.concatenate(
        [edge_index[0], jnp.zeros((pad,), edge_index.dtype)])
    dst_p = jnp.concatenate(
        [edge_index[1], jnp.full((pad,), _TRASH, edge_index.dtype)])

    ones = jnp.ones((_NC * _NP, _HALF), jnp.float32)
    gdeg = _prop(ones, src_p, dst_p)
    u3, dinv16, dinv216 = _init(x, fc_w, fc_b.reshape(1, _H), gdeg)
    u = u3.reshape(_NC * _NP, _HALF)

    w3 = W_w.reshape(3, _H, _H)
    wb = W_b.reshape(1, _H)
    layers = (
        (conv0_w1, conv0_w2, conv0_w3, bn0_gamma, bn0_beta),
        (conv1_w1, conv1_w2, conv1_w3, bn1_gamma, bn1_beta),
    )
    h = None
    for li, (w1, w2, w3c, gamma, beta) in enumerate(layers):
        outs, css, cqs = [], [], []
        cur = u
        for k, wk in enumerate((w1, w2, w3c)):
            g = _prop(cur, src_p, dst_p)
            if k < 2:
                o, un3, cs, cq = _hop(g, g, dinv16, dinv216, wk)
                cur = un3.reshape(_NC * _NP, _HALF)
            else:
                o, cs, cq = _hop_nou(g, g, dinv16, dinv216, wk)
            outs.append(o)
            css.append(cs)
            cqs.append(cq)
        args = (*outs, *css, *cqs, gamma.reshape(3, _H), beta.reshape(3, _H),
                w3, wb)
        if li == 0:
            h, un3 = _bn(*args, dinv16)
            u = un3.reshape(_NC * _NP, _HALF)
        else:
            (h,) = _bn_nou(*args)
    return h


# speedup vs baseline: 1.0513x; 1.0513x over previous
import jax
import jax.numpy as jnp
from jax import lax
from jax.experimental import pallas as pl
from jax.experimental.pallas import tpu as pltpu
from jax.experimental.pallas import tpu_sc as plsc

_N = 10000
_E = 160000
_D = 256
_H = 256
_HALF = 128
_EPS = 1e-5
_NC = 2
_NS = 16
_CH = 128
_EPAD = 163840
_PT_PROP = _EPAD // _NS
_NP = 10240
_RPT = _NP // _NS
_NACC = _NP
_TRASH = _N


_SC_CACHE = {}


def _sc_mesh():
    return plsc.VectorSubcoreMesh(
        core_axis_name="c", subcore_axis_name="s",
        num_cores=_NC, num_subcores=_NS)


_DEPTH = 2
_NCHUNK = _PT_PROP // _CH


def _prop_body(u_hbm, src_hbm, dst_hbm, g_hbm, idxs_v, idxd_v, rows_v, acc_sh,
               *sems):
    c = lax.axis_index("c")
    s = lax.axis_index("s")
    roff = c * _NP
    base = s * _PT_PROP

    def load_and_gather(j, b):
        off = base + j * _CH
        pltpu.sync_copy(src_hbm.at[pl.ds(off, _CH)], idxs_v.at[b])
        pltpu.sync_copy(dst_hbm.at[pl.ds(off, _CH)], idxd_v.at[b])
        for t in range(_CH // 16):
            idxs_v[b, pl.ds(t * 16, 16)] = idxs_v[b, pl.ds(t * 16, 16)] + roff
        pltpu.async_copy(u_hbm.at[idxs_v.at[b]], rows_v.at[b], sems[b])

    def finish(b):
        pltpu.make_async_copy(u_hbm.at[idxs_v.at[b]], rows_v.at[b],
                              sems[b]).wait()
        pltpu.sync_copy(rows_v.at[b], acc_sh.at[idxd_v.at[b]], add=True)

    for b in range(_DEPTH):
        load_and_gather(b, b)
    pltpu.sync_copy(u_hbm.at[pl.ds(roff + s * _RPT, _RPT)],
                    acc_sh.at[pl.ds(s * _RPT, _RPT)])
    plsc.subcore_barrier()

    def pair(jp, _):
        for b in range(_DEPTH):
            finish(b)
            load_and_gather(_DEPTH * jp + b + _DEPTH, b)
        return 0

    lax.fori_loop(0, _NCHUNK // _DEPTH - 1, pair, 0)
    for b in range(_DEPTH):
        finish(b)

    plsc.subcore_barrier()
    pltpu.sync_copy(acc_sh.at[pl.ds(s * _RPT, _RPT)],
                    g_hbm.at[c, pl.ds(s * _RPT, _RPT)])


def _prop(u, src_p, dst_p):
    if "prop" not in _SC_CACHE:
        _SC_CACHE["prop"] = pl.kernel(
            _prop_body,
            out_type=jax.ShapeDtypeStruct((_NC, _NP, _HALF), jnp.float32),
            mesh=_sc_mesh(),
            scratch_types=[
                pltpu.VMEM((_DEPTH, _CH), jnp.int32),
                pltpu.VMEM((_DEPTH, _CH), jnp.int32),
                pltpu.VMEM((_DEPTH, _CH, _HALF), jnp.float32),
                pltpu.VMEM_SHARED((_NACC, _HALF), jnp.float32),
            ] + [pltpu.SemaphoreType.DMA] * _DEPTH,
        )
    return _SC_CACHE["prop"](u, src_p, dst_p)


_R = 1000
_GRID = _N // _R


def _proj_body(x_ref, w_ref, b_ref, h_ref):
    h_ref[...] = jnp.dot(x_ref[...], w_ref[...],
                         preferred_element_type=jnp.float32) + b_ref[...]


_proj = pl.pallas_call(
    _proj_body,
    grid=(_GRID,),
    in_specs=[
        pl.BlockSpec((_R, _D), lambda i: (i, 0)),
        pl.BlockSpec((_D, _H), lambda i: (0, 0)),
        pl.BlockSpec((1, _H), lambda i: (0, 0)),
    ],
    out_specs=pl.BlockSpec((_R, _H), lambda i: (i, 0)),
    out_shape=jax.ShapeDtypeStruct((_N, _H), jnp.float32),
    compiler_params=pltpu.CompilerParams(dimension_semantics=("arbitrary",)),
)


def _init_body(h_ref, gdeg_ref, u_ref, dinv_ref, dinv2_ref):
    degsl = gdeg_ref[0]
    dinv128 = lax.rsqrt(degsl)
    dinv_ref[...] = dinv128[:, :16]
    dinv2_ref[...] = (1.0 / degsl)[:, :16]
    u = h_ref[...] * dinv128[:, 0:1]
    u_ref[0] = u[:, :_HALF]
    u_ref[1] = u[:, _HALF:]


_init = pl.pallas_call(
    _init_body,
    grid=(_GRID,),
    in_specs=[
        pl.BlockSpec((_R, _H), lambda i: (i, 0)),
        pl.BlockSpec((1, _R, _HALF), lambda i: (0, i, 0)),
    ],
    out_specs=[
        pl.BlockSpec((_NC, _R, _HALF), lambda i: (0, i, 0)),
        pl.BlockSpec((_R, 16), lambda i: (i, 0)),
        pl.BlockSpec((_R, 16), lambda i: (i, 0)),
    ],
    out_shape=[
        jax.ShapeDtypeStruct((_NC, _NP, _HALF), jnp.float32),
        jax.ShapeDtypeStruct((_N, 16), jnp.float32),
        jax.ShapeDtypeStruct((_N, 16), jnp.float32),
    ],
    compiler_params=pltpu.CompilerParams(dimension_semantics=("arbitrary",)),
)


def _hop_common(glo_ref, ghi_ref, dinv_ref, w_ref, out_ref, cs_ref, cq_ref):
    i = pl.program_id(0)
    g = jnp.concatenate([glo_ref[0], ghi_ref[0]], axis=1)
    y = g * dinv_ref[...][:, 0:1]
    o = jnp.dot(y, w_ref[...], preferred_element_type=jnp.float32)
    out_ref[...] = o

    @pl.when(i == 0)
    def _():
        cs_ref[...] = jnp.zeros_like(cs_ref)
        cq_ref[...] = jnp.zeros_like(cq_ref)

    cs_ref[...] += jnp.sum(o, axis=0, keepdims=True)
    cq_ref[...] += jnp.sum(o * o, axis=0, keepdims=True)
    return g


def _hop_body(glo_ref, ghi_ref, dinv_ref, dinv2_ref, w_ref, out_ref, un_ref,
              cs_ref, cq_ref):
    g = _hop_common(glo_ref, ghi_ref, dinv_ref, w_ref, out_ref, cs_ref, cq_ref)
    un = g * dinv2_ref[...][:, 0:1]
    un_ref[0] = un[:, :_HALF]
    un_ref[1] = un[:, _HALF:]


def _hop_body_nou(glo_ref, ghi_ref, dinv_ref, dinv2_ref, w_ref, out_ref,
                  cs_ref, cq_ref):
    _hop_common(glo_ref, ghi_ref, dinv_ref, w_ref, out_ref, cs_ref, cq_ref)


_HOP_IN_SPECS = [
    pl.BlockSpec((1, _R, _HALF), lambda i: (0, i, 0)),
    pl.BlockSpec((1, _R, _HALF), lambda i: (1, i, 0)),
    pl.BlockSpec((_R, 16), lambda i: (i, 0)),
    pl.BlockSpec((_R, 16), lambda i: (i, 0)),
    pl.BlockSpec((_H, _H), lambda i: (0, 0)),
]
_STAT_SPEC = pl.BlockSpec((1, _H), lambda i: (0, 0))
_STAT_SHAPE = jax.ShapeDtypeStruct((1, _H), jnp.float32)

_hop = pl.pallas_call(
    _hop_body,
    grid=(_GRID,),
    in_specs=_HOP_IN_SPECS,
    out_specs=[
        pl.BlockSpec((_R, _H), lambda i: (i, 0)),
        pl.BlockSpec((_NC, _R, _HALF), lambda i: (0, i, 0)),
        _STAT_SPEC,
        _STAT_SPEC,
    ],
    out_shape=[
        jax.ShapeDtypeStruct((_N, _H), jnp.float32),
        jax.ShapeDtypeStruct((_NC, _NP, _HALF), jnp.float32),
        _STAT_SHAPE,
        _STAT_SHAPE,
    ],
    compiler_params=pltpu.CompilerParams(dimension_semantics=("arbitrary",)),
)

_hop_nou = pl.pallas_call(
    _hop_body_nou,
    grid=(_GRID,),
    in_specs=_HOP_IN_SPECS,
    out_specs=[
        pl.BlockSpec((_R, _H), lambda i: (i, 0)),
        _STAT_SPEC,
        _STAT_SPEC,
    ],
    out_shape=[
        jax.ShapeDtypeStruct((_N, _H), jnp.float32),
        _STAT_SHAPE,
        _STAT_SHAPE,
    ],
    compiler_params=pltpu.CompilerParams(dimension_semantics=("arbitrary",)),
)


def _bn_common(o_refs, cs_refs, cq_refs, gam_ref, bet_ref, w3_ref, wb_ref):
    gam = gam_ref[...]
    bet = bet_ref[...]
    acc = jnp.zeros((_R, _H), jnp.float32) + wb_ref[...]
    for k in range(3):
        mu = cs_refs[k][...] / _N
        var = cq_refs[k][...] / _N - mu * mu
        sc = gam[k:k + 1] * lax.rsqrt(var + _EPS)
        shift = bet[k:k + 1] - mu * sc
        hn = o_refs[k][...] * sc + shift
        acc += jnp.dot(hn, w3_ref[k], preferred_element_type=jnp.float32)
    return jnp.maximum(acc, 0.0)


def _bn_body(o1, o2, o3, cs1, cs2, cs3, cq1, cq2, cq3, gam, bet, w3, wb,
             dinv_ref, h_ref, un_ref):
    h = _bn_common((o1, o2, o3), (cs1, cs2, cs3), (cq1, cq2, cq3), gam, bet,
                   w3, wb)
    h_ref[...] = h
    un = h * dinv_ref[...][:, 0:1]
    un_ref[0] = un[:, :_HALF]
    un_ref[1] = un[:, _HALF:]


def _bn_body_nou(o1, o2, o3, cs1, cs2, cs3, cq1, cq2, cq3, gam, bet, w3, wb,
                 h_ref):
    h_ref[...] = _bn_common((o1, o2, o3), (cs1, cs2, cs3), (cq1, cq2, cq3),
                            gam, bet, w3, wb)


_BN_IN_SPECS = (
    [pl.BlockSpec((_R, _H), lambda i: (i, 0))] * 3
    + [_STAT_SPEC] * 6
    + [
        pl.BlockSpec((3, _H), lambda i: (0, 0)),
        pl.BlockSpec((3, _H), lambda i: (0, 0)),
        pl.BlockSpec((3, _H, _H), lambda i: (0, 0, 0)),
        pl.BlockSpec((1, _H), lambda i: (0, 0)),
    ]
)

_bn = pl.pallas_call(
    _bn_body,
    grid=(_GRID,),
    in_specs=_BN_IN_SPECS + [pl.BlockSpec((_R, 16), lambda i: (i, 0))],
    out_specs=[
        pl.BlockSpec((_R, _H), lambda i: (i, 0)),
        pl.BlockSpec((_NC, _R, _HALF), lambda i: (0, i, 0)),
    ],
    out_shape=[
        jax.ShapeDtypeStruct((_N, _H), jnp.float32),
        jax.ShapeDtypeStruct((_NC, _NP, _HALF), jnp.float32),
    ],
    compiler_params=pltpu.CompilerParams(dimension_semantics=("arbitrary",)),
)

_bn_nou = pl.pallas_call(
    _bn_body_nou,
    grid=(_GRID,),
    in_specs=_BN_IN_SPECS,
    out_specs=[pl.BlockSpec((_R, _H), lambda i: (i, 0))],
    out_shape=[jax.ShapeDtypeStruct((_N, _H), jnp.float32)],
    compiler_params=pltpu.CompilerParams(dimension_semantics=("arbitrary",)),
)


def kernel(x, edge_index, fc_w, fc_b, conv0_w1, conv0_w2, conv0_w3, bn0_gamma,
           bn0_beta, conv1_w1, conv1_w2, conv1_w3, bn1_gamma, bn1_beta, W_w,
           W_b):
    pad = _EPAD - _E
    src_p = jnp.concatenate(
        [edge_index[0], jnp.zeros((pad,), edge_index.dtype)])
    dst_p = jnp.concatenate(
        [edge_index[1], jnp.full((pad,), _TRASH, edge_index.dtype)])

    ones = jnp.ones((_NC * _NP, _HALF), jnp.float32)
    gdeg = _prop(ones, src_p, dst_p)
    h0 = _proj(x, fc_w, fc_b.reshape(1, _H))
    u3, dinv16, dinv216 = _init(h0, gdeg)
    u = u3.reshape(_NC * _NP, _HALF)

    w3 = W_w.reshape(3, _H, _H)
    wb = W_b.reshape(1, _H)
    layers = (
        (conv0_w1, conv0_w2, conv0_w3, bn0_gamma, bn0_beta),
        (conv1_w1, conv1_w2, conv1_w3, bn1_gamma, bn1_beta),
    )
    h = None
    for li, (w1, w2, w3c, gamma, beta) in enumerate(layers):
        outs, css, cqs = [], [], []
        cur = u
        for k, wk in enumerate((w1, w2, w3c)):
            g = _prop(cur, src_p, dst_p)
            if k < 2:
                o, un3, cs, cq = _hop(g, g, dinv16, dinv216, wk)
                cur = un3.reshape(_NC * _NP, _HALF)
            else:
                o, cs, cq = _hop_nou(g, g, dinv16, dinv216, wk)
            outs.append(o)
            css.append(cs)
            cqs.append(cq)
        args = (*outs, *css, *cqs, gamma.reshape(3, _H), beta.reshape(3, _H),
                w3, wb)
        if li == 0:
            h, un3 = _bn(*args, dinv16)
            u = un3.reshape(_NC * _NP, _HALF)
        else:
            (h,) = _bn_nou(*args)
    return h


# speedup vs baseline: 1.1315x; 1.0763x over previous
import jax
import jax.numpy as jnp
from jax import lax
from jax.experimental import pallas as pl
from jax.experimental.pallas import tpu as pltpu
from jax.experimental.pallas import tpu_sc as plsc

_N = 10000
_E = 160000
_D = 256
_H = 256
_HALF = 128
_EPS = 1e-5
_NC = 2
_NS = 16
_CH = 128
_EPAD = 163840
_PT_PROP = _EPAD // _NS
_NP = 10240
_RPT = _NP // _NS
_NACC = _NP
_TRASH = _N


_SC_CACHE = {}


def _sc_mesh():
    return plsc.VectorSubcoreMesh(
        core_axis_name="c", subcore_axis_name="s",
        num_cores=_NC, num_subcores=_NS)


_DEPTH = 2
_IDEPTH = 4
_NCHUNK = _PT_PROP // _CH


def _prop_body(u_hbm, src_hbm, dst_hbm, g_hbm, idxs_v, idxd_v, rows_v, acc_sh,
               *sems):
    c = lax.axis_index("c")
    s = lax.axis_index("s")
    roff = c * _NP
    base = s * _PT_PROP

    def idx_load(j, q):
        off = base + j * _CH
        pltpu.async_copy(src_hbm.at[pl.ds(off, _CH)], idxs_v.at[q],
                         sems[_DEPTH + q])
        pltpu.async_copy(dst_hbm.at[pl.ds(off, _CH)], idxd_v.at[q],
                         sems[_DEPTH + q])

    def gather_issue(j, b, q):
        off = base + j * _CH
        pltpu.make_async_copy(src_hbm.at[pl.ds(off, _CH)], idxs_v.at[q],
                              sems[_DEPTH + q]).wait()
        pltpu.make_async_copy(dst_hbm.at[pl.ds(off, _CH)], idxd_v.at[q],
                              sems[_DEPTH + q]).wait()
        for t in range(_CH // 16):
            idxs_v[q, pl.ds(t * 16, 16)] = idxs_v[q, pl.ds(t * 16, 16)] + roff
        pltpu.async_copy(u_hbm.at[idxs_v.at[q]], rows_v.at[b], sems[b])

    def finish(b, q):
        pltpu.make_async_copy(u_hbm.at[idxs_v.at[q]], rows_v.at[b],
                              sems[b]).wait()
        pltpu.sync_copy(rows_v.at[b], acc_sh.at[idxd_v.at[q]], add=True)

    for q in range(_IDEPTH):
        idx_load(q, q)
    for b in range(_DEPTH):
        gather_issue(b, b, b)
    pltpu.sync_copy(u_hbm.at[pl.ds(roff + s * _RPT, _RPT)],
                    acc_sh.at[pl.ds(s * _RPT, _RPT)])
    plsc.subcore_barrier()

    def step(j, q, prefetch, issue):
        finish(q % _DEPTH, q)
        if prefetch:
            idx_load(j + _IDEPTH, q)
        if issue:
            gather_issue(j + _DEPTH, (q + _DEPTH) % _DEPTH,
                         (q + _DEPTH) % _IDEPTH)

    def quad(jp, _):
        for q in range(_IDEPTH):
            step(_IDEPTH * jp + q, q, True, True)
        return 0

    lax.fori_loop(0, _NCHUNK // _IDEPTH - 1, quad, 0)
    for q in range(_IDEPTH):
        j = _NCHUNK - _IDEPTH + q
        step(j, q, False, j + _DEPTH < _NCHUNK)

    plsc.subcore_barrier()
    pltpu.sync_copy(acc_sh.at[pl.ds(s * _RPT, _RPT)],
                    g_hbm.at[c, pl.ds(s * _RPT, _RPT)])


def _prop(u, src_p, dst_p):
    if "prop" not in _SC_CACHE:
        _SC_CACHE["prop"] = pl.kernel(
            _prop_body,
            out_type=jax.ShapeDtypeStruct((_NC, _NP, _HALF), jnp.float32),
            mesh=_sc_mesh(),
            scratch_types=[
                pltpu.VMEM((_IDEPTH, _CH), jnp.int32),
                pltpu.VMEM((_IDEPTH, _CH), jnp.int32),
                pltpu.VMEM((_DEPTH, _CH, _HALF), jnp.float32),
                pltpu.VMEM_SHARED((_NACC, _HALF), jnp.float32),
            ] + [pltpu.SemaphoreType.DMA] * (_DEPTH + _IDEPTH),
        )
    return _SC_CACHE["prop"](u, src_p, dst_p)


_R = 1000
_GRID = _N // _R


def _proj_body(x_ref, w_ref, b_ref, h_ref):
    h_ref[...] = jnp.dot(x_ref[...], w_ref[...],
                         preferred_element_type=jnp.float32) + b_ref[...]


_proj = pl.pallas_call(
    _proj_body,
    grid=(_GRID,),
    in_specs=[
        pl.BlockSpec((_R, _D), lambda i: (i, 0)),
        pl.BlockSpec((_D, _H), lambda i: (0, 0)),
        pl.BlockSpec((1, _H), lambda i: (0, 0)),
    ],
    out_specs=pl.BlockSpec((_R, _H), lambda i: (i, 0)),
    out_shape=jax.ShapeDtypeStruct((_N, _H), jnp.float32),
    compiler_params=pltpu.CompilerParams(dimension_semantics=("arbitrary",)),
)


def _init_body(h_ref, gdeg_ref, u_ref, dinv_ref, dinv2_ref):
    degsl = gdeg_ref[0]
    dinv128 = lax.rsqrt(degsl)
    dinv_ref[...] = dinv128[:, :16]
    dinv2_ref[...] = (1.0 / degsl)[:, :16]
    u = h_ref[...] * dinv128[:, 0:1]
    u_ref[0] = u[:, :_HALF]
    u_ref[1] = u[:, _HALF:]


_init = pl.pallas_call(
    _init_body,
    grid=(_GRID,),
    in_specs=[
        pl.BlockSpec((_R, _H), lambda i: (i, 0)),
        pl.BlockSpec((1, _R, _HALF), lambda i: (0, i, 0)),
    ],
    out_specs=[
        pl.BlockSpec((_NC, _R, _HALF), lambda i: (0, i, 0)),
        pl.BlockSpec((_R, 16), lambda i: (i, 0)),
        pl.BlockSpec((_R, 16), lambda i: (i, 0)),
    ],
    out_shape=[
        jax.ShapeDtypeStruct((_NC, _NP, _HALF), jnp.float32),
        jax.ShapeDtypeStruct((_N, 16), jnp.float32),
        jax.ShapeDtypeStruct((_N, 16), jnp.float32),
    ],
    compiler_params=pltpu.CompilerParams(dimension_semantics=("arbitrary",)),
)


def _hop_common(glo_ref, ghi_ref, dinv_ref, w_ref, out_ref, cs_ref, cq_ref):
    i = pl.program_id(0)
    g = jnp.concatenate([glo_ref[0], ghi_ref[0]], axis=1)
    y = g * dinv_ref[...][:, 0:1]
    o = jnp.dot(y, w_ref[...], preferred_element_type=jnp.float32)
    out_ref[...] = o

    @pl.when(i == 0)
    def _():
        cs_ref[...] = jnp.zeros_like(cs_ref)
        cq_ref[...] = jnp.zeros_like(cq_ref)

    cs_ref[...] += jnp.sum(o, axis=0, keepdims=True)
    cq_ref[...] += jnp.sum(o * o, axis=0, keepdims=True)
    return g


def _hop_body(glo_ref, ghi_ref, dinv_ref, dinv2_ref, w_ref, out_ref, un_ref,
              cs_ref, cq_ref):
    g = _hop_common(glo_ref, ghi_ref, dinv_ref, w_ref, out_ref, cs_ref, cq_ref)
    un = g * dinv2_ref[...][:, 0:1]
    un_ref[0] = un[:, :_HALF]
    un_ref[1] = un[:, _HALF:]


def _hop_body_nou(glo_ref, ghi_ref, dinv_ref, dinv2_ref, w_ref, out_ref,
                  cs_ref, cq_ref):
    _hop_common(glo_ref, ghi_ref, dinv_ref, w_ref, out_ref, cs_ref, cq_ref)


_HOP_IN_SPECS = [
    pl.BlockSpec((1, _R, _HALF), lambda i: (0, i, 0)),
    pl.BlockSpec((1, _R, _HALF), lambda i: (1, i, 0)),
    pl.BlockSpec((_R, 16), lambda i: (i, 0)),
    pl.BlockSpec((_R, 16), lambda i: (i, 0)),
    pl.BlockSpec((_H, _H), lambda i: (0, 0)),
]
_STAT_SPEC = pl.BlockSpec((1, _H), lambda i: (0, 0))
_STAT_SHAPE = jax.ShapeDtypeStruct((1, _H), jnp.float32)

_hop = pl.pallas_call(
    _hop_body,
    grid=(_GRID,),
    in_specs=_HOP_IN_SPECS,
    out_specs=[
        pl.BlockSpec((_R, _H), lambda i: (i, 0)),
        pl.BlockSpec((_NC, _R, _HALF), lambda i: (0, i, 0)),
        _STAT_SPEC,
        _STAT_SPEC,
    ],
    out_shape=[
        jax.ShapeDtypeStruct((_N, _H), jnp.float32),
        jax.ShapeDtypeStruct((_NC, _NP, _HALF), jnp.float32),
        _STAT_SHAPE,
        _STAT_SHAPE,
    ],
    compiler_params=pltpu.CompilerParams(dimension_semantics=("arbitrary",)),
)

_hop_nou = pl.pallas_call(
    _hop_body_nou,
    grid=(_GRID,),
    in_specs=_HOP_IN_SPECS,
    out_specs=[
        pl.BlockSpec((_R, _H), lambda i: (i, 0)),
        _STAT_SPEC,
        _STAT_SPEC,
    ],
    out_shape=[
        jax.ShapeDtypeStruct((_N, _H), jnp.float32),
        _STAT_SHAPE,
        _STAT_SHAPE,
    ],
    compiler_params=pltpu.CompilerParams(dimension_semantics=("arbitrary",)),
)


def _bn_common(o_refs, cs_refs, cq_refs, gam_ref, bet_ref, w3_ref, wb_ref):
    gam = gam_ref[...]
    bet = bet_ref[...]
    acc = jnp.zeros((_R, _H), jnp.float32) + wb_ref[...]
    for k in range(3):
        mu = cs_refs[k][...] / _N
        var = cq_refs[k][...] / _N - mu * mu
        sc = gam[k:k + 1] * lax.rsqrt(var + _EPS)
        shift = bet[k:k + 1] - mu * sc
        hn = o_refs[k][...] * sc + shift
        acc += jnp.dot(hn, w3_ref[k], preferred_element_type=jnp.float32)
    return jnp.maximum(acc, 0.0)


def _bn_body(o1, o2, o3, cs1, cs2, cs3, cq1, cq2, cq3, gam, bet, w3, wb,
             dinv_ref, h_ref, un_ref):
    h = _bn_common((o1, o2, o3), (cs1, cs2, cs3), (cq1, cq2, cq3), gam, bet,
                   w3, wb)
    h_ref[...] = h
    un = h * dinv_ref[...][:, 0:1]
    un_ref[0] = un[:, :_HALF]
    un_ref[1] = un[:, _HALF:]


def _bn_body_nou(o1, o2, o3, cs1, cs2, cs3, cq1, cq2, cq3, gam, bet, w3, wb,
                 h_ref):
    h_ref[...] = _bn_common((o1, o2, o3), (cs1, cs2, cs3), (cq1, cq2, cq3),
                            gam, bet, w3, wb)


_BN_IN_SPECS = (
    [pl.BlockSpec((_R, _H), lambda i: (i, 0))] * 3
    + [_STAT_SPEC] * 6
    + [
        pl.BlockSpec((3, _H), lambda i: (0, 0)),
        pl.BlockSpec((3, _H), lambda i: (0, 0)),
        pl.BlockSpec((3, _H, _H), lambda i: (0, 0, 0)),
        pl.BlockSpec((1, _H), lambda i: (0, 0)),
    ]
)

_bn = pl.pallas_call(
    _bn_body,
    grid=(_GRID,),
    in_specs=_BN_IN_SPECS + [pl.BlockSpec((_R, 16), lambda i: (i, 0))],
    out_specs=[
        pl.BlockSpec((_R, _H), lambda i: (i, 0)),
        pl.BlockSpec((_NC, _R, _HALF), lambda i: (0, i, 0)),
    ],
    out_shape=[
        jax.ShapeDtypeStruct((_N, _H), jnp.float32),
        jax.ShapeDtypeStruct((_NC, _NP, _HALF), jnp.float32),
    ],
    compiler_params=pltpu.CompilerParams(dimension_semantics=("arbitrary",)),
)

_bn_nou = pl.pallas_call(
    _bn_body_nou,
    grid=(_GRID,),
    in_specs=_BN_IN_SPECS,
    out_specs=[pl.BlockSpec((_R, _H), lambda i: (i, 0))],
    out_shape=[jax.ShapeDtypeStruct((_N, _H), jnp.float32)],
    compiler_params=pltpu.CompilerParams(dimension_semantics=("arbitrary",)),
)


def kernel(x, edge_index, fc_w, fc_b, conv0_w1, conv0_w2, conv0_w3, bn0_gamma,
           bn0_beta, conv1_w1, conv1_w2, conv1_w3, bn1_gamma, bn1_beta, W_w,
           W_b):
    pad = _EPAD - _E
    src_p = jnp.concatenate(
        [edge_index[0], jnp.zeros((pad,), edge_index.dtype)])
    dst_p = jnp.concatenate(
        [edge_index[1], jnp.full((pad,), _TRASH, edge_index.dtype)])

    ones = jnp.ones((_NC * _NP, _HALF), jnp.float32)
    gdeg = _prop(ones, src_p, dst_p)
    h0 = _proj(x, fc_w, fc_b.reshape(1, _H))
    u3, dinv16, dinv216 = _init(h0, gdeg)
    u = u3.reshape(_NC * _NP, _HALF)

    w3 = W_w.reshape(3, _H, _H)
    wb = W_b.reshape(1, _H)
    layers = (
        (conv0_w1, conv0_w2, conv0_w3, bn0_gamma, bn0_beta),
        (conv1_w1, conv1_w2, conv1_w3, bn1_gamma, bn1_beta),
    )
    h = None
    for li, (w1, w2, w3c, gamma, beta) in enumerate(layers):
        outs, css, cqs = [], [], []
        cur = u
        for k, wk in enumerate((w1, w2, w3c)):
            g = _prop(cur, src_p, dst_p)
            if k < 2:
                o, un3, cs, cq = _hop(g, g, dinv16, dinv216, wk)
                cur = un3.reshape(_NC * _NP, _HALF)
            else:
                o, cs, cq = _hop_nou(g, g, dinv16, dinv216, wk)
            outs.append(o)
            css.append(cs)
            cqs.append(cq)
        args = (*outs, *css, *cqs, gamma.reshape(3, _H), beta.reshape(3, _H),
                w3, wb)
        if li == 0:
            h, un3 = _bn(*args, dinv16)
            u = un3.reshape(_NC * _NP, _HALF)
        else:
            (h,) = _bn_nou(*args)
    return h


# speedup vs baseline: 1.3111x; 1.1587x over previous
import jax
import jax.numpy as jnp
from jax import lax
from jax.experimental import pallas as pl
from jax.experimental.pallas import tpu as pltpu
from jax.experimental.pallas import tpu_sc as plsc

_N = 10000
_E = 160000
_D = 256
_H = 256
_HALF = 128
_EPS = 1e-5
_NC = 2
_NS = 16
_CH = 128
_EPAD = 163840
_PT_PROP = _EPAD // _NS
_NP = 10240
_RPT = _NP // _NS
_NACC = _NP
_TRASH = _N


_SC_CACHE = {}


def _sc_mesh():
    return plsc.VectorSubcoreMesh(
        core_axis_name="c", subcore_axis_name="s",
        num_cores=_NC, num_subcores=_NS)


_DEPTH = 2
_IDEPTH = 4
_NCHUNK = _PT_PROP // _CH


def _prop_body(u_hbm, src_hbm, dst_hbm, g_hbm, idxs_v, idxd_v, rows_v, acc_sh,
               *sems):
    c = lax.axis_index("c")
    s = lax.axis_index("s")
    roff = c * _NP
    base = s * _PT_PROP

    def idx_load(j, q):
        off = base + j * _CH
        pltpu.async_copy(src_hbm.at[pl.ds(off, _CH)], idxs_v.at[q],
                         sems[_DEPTH + q])
        pltpu.async_copy(dst_hbm.at[pl.ds(off, _CH)], idxd_v.at[q],
                         sems[_DEPTH + q])

    def gather_issue(j, b, q):
        off = base + j * _CH
        pltpu.make_async_copy(src_hbm.at[pl.ds(off, _CH)], idxs_v.at[q],
                              sems[_DEPTH + q]).wait()
        pltpu.make_async_copy(dst_hbm.at[pl.ds(off, _CH)], idxd_v.at[q],
                              sems[_DEPTH + q]).wait()
        for t in range(_CH // 16):
            idxs_v[q, pl.ds(t * 16, 16)] = idxs_v[q, pl.ds(t * 16, 16)] + roff
        pltpu.async_copy(u_hbm.at[idxs_v.at[q]], rows_v.at[b], sems[b])

    def finish(b, q):
        pltpu.make_async_copy(u_hbm.at[idxs_v.at[q]], rows_v.at[b],
                              sems[b]).wait()
        pltpu.sync_copy(rows_v.at[b], acc_sh.at[idxd_v.at[q]], add=True)

    for q in range(_IDEPTH):
        idx_load(q, q)
    for b in range(_DEPTH):
        gather_issue(b, b, b)
    pltpu.sync_copy(u_hbm.at[pl.ds(roff + s * _RPT, _RPT)],
                    acc_sh.at[pl.ds(s * _RPT, _RPT)])
    plsc.subcore_barrier()

    def step(j, q, prefetch, issue):
        finish(q % _DEPTH, q)
        if prefetch:
            idx_load(j + _IDEPTH, q)
        if issue:
            gather_issue(j + _DEPTH, (q + _DEPTH) % _DEPTH,
                         (q + _DEPTH) % _IDEPTH)

    def quad(jp, _):
        for q in range(_IDEPTH):
            step(_IDEPTH * jp + q, q, True, True)
        return 0

    lax.fori_loop(0, _NCHUNK // _IDEPTH - 1, quad, 0)
    for q in range(_IDEPTH):
        j = _NCHUNK - _IDEPTH + q
        step(j, q, False, j + _DEPTH < _NCHUNK)

    plsc.subcore_barrier()
    pltpu.sync_copy(acc_sh.at[pl.ds(s * _RPT, _RPT)],
                    g_hbm.at[c, pl.ds(s * _RPT, _RPT)])


_PT_DEG = _EPAD // _NC // _NS
_NCH_DEG = _PT_DEG // _CH


def _deg_body(ones_hbm, dst_hbm, g_hbm, idxd_v, ones_v, acc_sh, *sems):
    c = lax.axis_index("c")
    s = lax.axis_index("s")
    base = c * (_EPAD // _NC) + s * _PT_DEG

    def idx_load(j, q):
        pltpu.async_copy(dst_hbm.at[pl.ds(base + j * _CH, _CH)],
                         idxd_v.at[q], sems[q])

    pltpu.sync_copy(ones_hbm, ones_v)
    for q in range(_IDEPTH):
        idx_load(q, q)
    for r in range(_RPT // _CH):
        pltpu.sync_copy(ones_v, acc_sh.at[pl.ds(s * _RPT + r * _CH, _CH)])
    plsc.subcore_barrier()

    def step(j, q, prefetch):
        pltpu.make_async_copy(dst_hbm.at[pl.ds(base + j * _CH, _CH)],
                              idxd_v.at[q], sems[q]).wait()
        pltpu.sync_copy(ones_v, acc_sh.at[idxd_v.at[q]], add=True)
        if prefetch:
            idx_load(j + _IDEPTH, q)

    def quad(jp, _):
        for q in range(_IDEPTH):
            step(_IDEPTH * jp + q, q, True)
        return 0

    lax.fori_loop(0, _NCH_DEG // _IDEPTH - 1, quad, 0)
    for q in range(_IDEPTH):
        step(_NCH_DEG - _IDEPTH + q, q, False)

    plsc.subcore_barrier()
    pltpu.sync_copy(acc_sh.at[pl.ds(s * _RPT, _RPT)],
                    g_hbm.at[c, pl.ds(s * _RPT, _RPT)])


def _deg(ones128, dst_p):
    if "deg" not in _SC_CACHE:
        _SC_CACHE["deg"] = pl.kernel(
            _deg_body,
            out_type=jax.ShapeDtypeStruct((_NC, _NP, _HALF), jnp.float32),
            mesh=_sc_mesh(),
            scratch_types=[
                pltpu.VMEM((_IDEPTH, _CH), jnp.int32),
                pltpu.VMEM((_CH, _HALF), jnp.float32),
                pltpu.VMEM_SHARED((_NACC, _HALF), jnp.float32),
            ] + [pltpu.SemaphoreType.DMA] * _IDEPTH,
        )
    return _SC_CACHE["deg"](ones128, dst_p)


def _prop(u, src_p, dst_p):
    if "prop" not in _SC_CACHE:
        _SC_CACHE["prop"] = pl.kernel(
            _prop_body,
            out_type=jax.ShapeDtypeStruct((_NC, _NP, _HALF), jnp.float32),
            mesh=_sc_mesh(),
            scratch_types=[
                pltpu.VMEM((_IDEPTH, _CH), jnp.int32),
                pltpu.VMEM((_IDEPTH, _CH), jnp.int32),
                pltpu.VMEM((_DEPTH, _CH, _HALF), jnp.float32),
                pltpu.VMEM_SHARED((_NACC, _HALF), jnp.float32),
            ] + [pltpu.SemaphoreType.DMA] * (_DEPTH + _IDEPTH),
        )
    return _SC_CACHE["prop"](u, src_p, dst_p)


_R = 1000
_GRID = _N // _R


def _proj_body(x_ref, w_ref, b_ref, h_ref):
    h_ref[...] = jnp.dot(x_ref[...], w_ref[...],
                         preferred_element_type=jnp.float32) + b_ref[...]


_proj = pl.pallas_call(
    _proj_body,
    grid=(_GRID,),
    in_specs=[
        pl.BlockSpec((_R, _D), lambda i: (i, 0)),
        pl.BlockSpec((_D, _H), lambda i: (0, 0)),
        pl.BlockSpec((1, _H), lambda i: (0, 0)),
    ],
    out_specs=pl.BlockSpec((_R, _H), lambda i: (i, 0)),
    out_shape=jax.ShapeDtypeStruct((_N, _H), jnp.float32),
    compiler_params=pltpu.CompilerParams(dimension_semantics=("arbitrary",)),
)


def _init_body(h_ref, gdeg_ref, u_ref, dinv_ref, dinv2_ref):
    degsl = gdeg_ref[0] + gdeg_ref[1] - 1.0
    dinv128 = lax.rsqrt(degsl)
    dinv_ref[...] = dinv128[:, :16]
    dinv2_ref[...] = (1.0 / degsl)[:, :16]
    u = h_ref[...] * dinv128[:, 0:1]
    u_ref[0] = u[:, :_HALF]
    u_ref[1] = u[:, _HALF:]


_init = pl.pallas_call(
    _init_body,
    grid=(_GRID,),
    in_specs=[
        pl.BlockSpec((_R, _H), lambda i: (i, 0)),
        pl.BlockSpec((_NC, _R, _HALF), lambda i: (0, i, 0)),
    ],
    out_specs=[
        pl.BlockSpec((_NC, _R, _HALF), lambda i: (0, i, 0)),
        pl.BlockSpec((_R, 16), lambda i: (i, 0)),
        pl.BlockSpec((_R, 16), lambda i: (i, 0)),
    ],
    out_shape=[
        jax.ShapeDtypeStruct((_NC, _NP, _HALF), jnp.float32),
        jax.ShapeDtypeStruct((_N, 16), jnp.float32),
        jax.ShapeDtypeStruct((_N, 16), jnp.float32),
    ],
    compiler_params=pltpu.CompilerParams(dimension_semantics=("arbitrary",)),
)


def _hop_common(glo_ref, ghi_ref, dinv_ref, w_ref, out_ref, cs_ref, cq_ref):
    i = pl.program_id(0)
    g = jnp.concatenate([glo_ref[0], ghi_ref[0]], axis=1)
    y = g * dinv_ref[...][:, 0:1]
    o = jnp.dot(y, w_ref[...], preferred_element_type=jnp.float32)
    out_ref[...] = o

    @pl.when(i == 0)
    def _():
        cs_ref[...] = jnp.zeros_like(cs_ref)
        cq_ref[...] = jnp.zeros_like(cq_ref)

    cs_ref[...] += jnp.sum(o, axis=0, keepdims=True)
    cq_ref[...] += jnp.sum(o * o, axis=0, keepdims=True)
    return g


def _hop_body(glo_ref, ghi_ref, dinv_ref, dinv2_ref, w_ref, out_ref, un_ref,
              cs_ref, cq_ref):
    g = _hop_common(glo_ref, ghi_ref, dinv_ref, w_ref, out_ref, cs_ref, cq_ref)
    un = g * dinv2_ref[...][:, 0:1]
    un_ref[0] = un[:, :_HALF]
    un_ref[1] = un[:, _HALF:]


def _hop_body_nou(glo_ref, ghi_ref, dinv_ref, dinv2_ref, w_ref, out_ref,
                  cs_ref, cq_ref):
    _hop_common(glo_ref, ghi_ref, dinv_ref, w_ref, out_ref, cs_ref, cq_ref)


_HOP_IN_SPECS = [
    pl.BlockSpec((1, _R, _HALF), lambda i: (0, i, 0)),
    pl.BlockSpec((1, _R, _HALF), lambda i: (1, i, 0)),
    pl.BlockSpec((_R, 16), lambda i: (i, 0)),
    pl.BlockSpec((_R, 16), lambda i: (i, 0)),
    pl.BlockSpec((_H, _H), lambda i: (0, 0)),
]
_STAT_SPEC = pl.BlockSpec((1, _H), lambda i: (0, 0))
_STAT_SHAPE = jax.ShapeDtypeStruct((1, _H), jnp.float32)

_hop = pl.pallas_call(
    _hop_body,
    grid=(_GRID,),
    in_specs=_HOP_IN_SPECS,
    out_specs=[
        pl.BlockSpec((_R, _H), lambda i: (i, 0)),
        pl.BlockSpec((_NC, _R, _HALF), lambda i: (0, i, 0)),
        _STAT_SPEC,
        _STAT_SPEC,
    ],
    out_shape=[
        jax.ShapeDtypeStruct((_N, _H), jnp.float32),
        jax.ShapeDtypeStruct((_NC, _NP, _HALF), jnp.float32),
        _STAT_SHAPE,
        _STAT_SHAPE,
    ],
    compiler_params=pltpu.CompilerParams(dimension_semantics=("arbitrary",)),
)

_hop_nou = pl.pallas_call(
    _hop_body_nou,
    grid=(_GRID,),
    in_specs=_HOP_IN_SPECS,
    out_specs=[
        pl.BlockSpec((_R, _H), lambda i: (i, 0)),
        _STAT_SPEC,
        _STAT_SPEC,
    ],
    out_shape=[
        jax.ShapeDtypeStruct((_N, _H), jnp.float32),
        _STAT_SHAPE,
        _STAT_SHAPE,
    ],
    compiler_params=pltpu.CompilerParams(dimension_semantics=("arbitrary",)),
)


def _bn_common(o_refs, cs_refs, cq_refs, gam_ref, bet_ref, w3_ref, wb_ref):
    gam = gam_ref[...]
    bet = bet_ref[...]
    acc = jnp.zeros((_R, _H), jnp.float32) + wb_ref[...]
    for k in range(3):
        mu = cs_refs[k][...] / _N
        var = cq_refs[k][...] / _N - mu * mu
        sc = gam[k:k + 1] * lax.rsqrt(var + _EPS)
        shift = bet[k:k + 1] - mu * sc
        hn = o_refs[k][...] * sc + shift
        acc += jnp.dot(hn, w3_ref[k], preferred_element_type=jnp.float32)
    return jnp.maximum(acc, 0.0)


def _bn_body(o1, o2, o3, cs1, cs2, cs3, cq1, cq2, cq3, gam, bet, w3, wb,
             dinv_ref, h_ref, un_ref):
    h = _bn_common((o1, o2, o3), (cs1, cs2, cs3), (cq1, cq2, cq3), gam, bet,
                   w3, wb)
    h_ref[...] = h
    un = h * dinv_ref[...][:, 0:1]
    un_ref[0] = un[:, :_HALF]
    un_ref[1] = un[:, _HALF:]


def _bn_body_nou(o1, o2, o3, cs1, cs2, cs3, cq1, cq2, cq3, gam, bet, w3, wb,
                 h_ref):
    h_ref[...] = _bn_common((o1, o2, o3), (cs1, cs2, cs3), (cq1, cq2, cq3),
                            gam, bet, w3, wb)


_BN_IN_SPECS = (
    [pl.BlockSpec((_R, _H), lambda i: (i, 0))] * 3
    + [_STAT_SPEC] * 6
    + [
        pl.BlockSpec((3, _H), lambda i: (0, 0)),
        pl.BlockSpec((3, _H), lambda i: (0, 0)),
        pl.BlockSpec((3, _H, _H), lambda i: (0, 0, 0)),
        pl.BlockSpec((1, _H), lambda i: (0, 0)),
    ]
)

_bn = pl.pallas_call(
    _bn_body,
    grid=(_GRID,),
    in_specs=_BN_IN_SPECS + [pl.BlockSpec((_R, 16), lambda i: (i, 0))],
    out_specs=[
        pl.BlockSpec((_R, _H), lambda i: (i, 0)),
        pl.BlockSpec((_NC, _R, _HALF), lambda i: (0, i, 0)),
    ],
    out_shape=[
        jax.ShapeDtypeStruct((_N, _H), jnp.float32),
        jax.ShapeDtypeStruct((_NC, _NP, _HALF), jnp.float32),
    ],
    compiler_params=pltpu.CompilerParams(dimension_semantics=("arbitrary",)),
)

_bn_nou = pl.pallas_call(
    _bn_body_nou,
    grid=(_GRID,),
    in_specs=_BN_IN_SPECS,
    out_specs=[pl.BlockSpec((_R, _H), lambda i: (i, 0))],
    out_shape=[jax.ShapeDtypeStruct((_N, _H), jnp.float32)],
    compiler_params=pltpu.CompilerParams(dimension_semantics=("arbitrary",)),
)


def kernel(x, edge_index, fc_w, fc_b, conv0_w1, conv0_w2, conv0_w3, bn0_gamma,
           bn0_beta, conv1_w1, conv1_w2, conv1_w3, bn1_gamma, bn1_beta, W_w,
           W_b):
    pad = _EPAD - _E
    src_p = jnp.concatenate(
        [edge_index[0], jnp.zeros((pad,), edge_index.dtype)])
    dst_p = jnp.concatenate(
        [edge_index[1], jnp.full((pad,), _TRASH, edge_index.dtype)])

    ones128 = jnp.ones((_CH, _HALF), jnp.float32)
    gdeg = _deg(ones128, dst_p)
    h0 = _proj(x, fc_w, fc_b.reshape(1, _H))
    u3, dinv16, dinv216 = _init(h0, gdeg)
    u = u3.reshape(_NC * _NP, _HALF)

    w3 = W_w.reshape(3, _H, _H)
    wb = W_b.reshape(1, _H)
    layers = (
        (conv0_w1, conv0_w2, conv0_w3, bn0_gamma, bn0_beta),
        (conv1_w1, conv1_w2, conv1_w3, bn1_gamma, bn1_beta),
    )
    h = None
    for li, (w1, w2, w3c, gamma, beta) in enumerate(layers):
        outs, css, cqs = [], [], []
        cur = u
        for k, wk in enumerate((w1, w2, w3c)):
            g = _prop(cur, src_p, dst_p)
            if k < 2:
                o, un3, cs, cq = _hop(g, g, dinv16, dinv216, wk)
                cur = un3.reshape(_NC * _NP, _HALF)
            else:
                o, cs, cq = _hop_nou(g, g, dinv16, dinv216, wk)
            outs.append(o)
            css.append(cs)
            cqs.append(cq)
        args = (*outs, *css, *cqs, gamma.reshape(3, _H), beta.reshape(3, _H),
                w3, wb)
        if li == 0:
            h, un3 = _bn(*args, dinv16)
            u = un3.reshape(_NC * _NP, _HALF)
        else:
            (h,) = _bn_nou(*args)
    return h
